# baseline (speedup 1.0000x reference)
import functools

import jax
import jax.numpy as jnp
from jax import lax
from jax.experimental import pallas as pl
from jax.experimental.pallas import tpu as pltpu
from jax.experimental.pallas import tpu_sc as plsc

BATCH = 16384
DIM = 32
NW = 32
BPW = BATCH // NW
NCHUNK = 4
CHUNK = BPW // NCHUNK


def _bpr_body(user_h, item_i_h, item_j_h, Ut_h, It_h, out_h,
              uidx, iidx, jidx, urows, iirows, ijrows, outv, sem):
    cid = lax.axis_index("c")
    sid = lax.axis_index("s")
    wid = sid * 2 + cid
    base = wid * BPW

    pltpu.sync_copy(user_h.at[wid], uidx)
    pltpu.sync_copy(item_i_h.at[wid], iidx)
    pltpu.sync_copy(item_j_h.at[wid], jidx)

    def d_step(d, _):
        copies = []
        for ch in range(NCHUNK):
            dst = pl.ds(ch * CHUNK, CHUNK)
            copies.append(
                pltpu.async_copy(Ut_h.at[d].at[uidx.at[ch]], urows.at[d].at[dst], sem))
            copies.append(
                pltpu.async_copy(It_h.at[d].at[iidx.at[ch]], iirows.at[d].at[dst], sem))
            copies.append(
                pltpu.async_copy(It_h.at[d].at[jidx.at[ch]], ijrows.at[d].at[dst], sem))
        for c in copies:
            c.wait()
        return 0

    lax.fori_loop(0, DIM, d_step, 0)

    def group_body(g, _):
        b16 = pl.ds(g * 16, 16)

        def d_step2(d, acc):
            u = urows[d, b16]
            ei = iirows[d, b16]
            ej = ijrows[d, b16]
            return acc + u * (ei - ej)

        acc = lax.fori_loop(0, DIM, d_step2, jnp.zeros((16,), jnp.float32))
        outv[b16] = acc
        return 0

    lax.fori_loop(0, BPW // 16, group_body, 0)

    pltpu.sync_copy(outv, out_h.at[pl.ds(base, BPW)])


@jax.jit
def kernel(user, item_i, item_j, U, I):
    user3 = user.astype(jnp.int32).reshape(NW, NCHUNK, CHUNK)
    item_i3 = item_i.astype(jnp.int32).reshape(NW, NCHUNK, CHUNK)
    item_j3 = item_j.astype(jnp.int32).reshape(NW, NCHUNK, CHUNK)

    mesh = plsc.VectorSubcoreMesh(core_axis_name="c", subcore_axis_name="s")
    f = functools.partial(
        pl.kernel,
        out_type=jax.ShapeDtypeStruct((BATCH,), jnp.float32),
        mesh=mesh,
        compiler_params=pltpu.CompilerParams(
            needs_layout_passes=False, use_tc_tiling_on_sc=False
        ),
        scratch_types=[
            pltpu.VMEM((NCHUNK, CHUNK), jnp.int32),
            pltpu.VMEM((NCHUNK, CHUNK), jnp.int32),
            pltpu.VMEM((NCHUNK, CHUNK), jnp.int32),
            pltpu.VMEM((DIM, BPW), jnp.float32),
            pltpu.VMEM((DIM, BPW), jnp.float32),
            pltpu.VMEM((DIM, BPW), jnp.float32),
            pltpu.VMEM((BPW,), jnp.float32),
            pltpu.SemaphoreType.DMA,
        ],
    )(_bpr_body)
    return f(user3, item_i3, item_j3, U.T, I.T)


# speedup vs baseline: 5.7437x; 5.7437x over previous
import functools

import jax
import jax.numpy as jnp
from jax import lax
from jax.experimental import pallas as pl
from jax.experimental.pallas import tpu as pltpu
from jax.experimental.pallas import tpu_sc as plsc

BATCH = 16384
DIM = 32
NW = 32
BPW = BATCH // NW
NCHUNK = 4
CHUNK = BPW // NCHUNK


def _bpr_body(user_h, item_i_h, item_j_h, U_h, I_h, out_h,
              uidx, iidx, jidx, urows, iirows, ijrows, outv, sem):
    cid = lax.axis_index("c")
    sid = lax.axis_index("s")
    wid = sid * 2 + cid
    base = wid * BPW

    pltpu.sync_copy(user_h.at[wid], uidx)
    pltpu.sync_copy(item_i_h.at[wid], iidx)
    pltpu.sync_copy(item_j_h.at[wid], jidx)

    copies = []
    for ch in range(NCHUNK):
        dst = pl.ds(ch * CHUNK, CHUNK)
        copies.append(pltpu.async_copy(U_h.at[uidx.at[ch]], urows.at[dst], sem))
        copies.append(pltpu.async_copy(I_h.at[iidx.at[ch]], iirows.at[dst], sem))
        copies.append(pltpu.async_copy(I_h.at[jidx.at[ch]], ijrows.at[dst], sem))
    for c in copies:
        c.wait()

    lanes = lax.iota(jnp.int32, 16)

    def group_body(g, _):
        row = g * 16 + lanes

        def d_step(d, acc):
            col = (lanes + d) & (DIM - 1)
            u = plsc.load_gather(urows, [row, col])
            ei = plsc.load_gather(iirows, [row, col])
            ej = plsc.load_gather(ijrows, [row, col])
            return acc + u * (ei - ej)

        acc = lax.fori_loop(0, DIM, d_step, jnp.zeros((16,), jnp.float32))
        outv[pl.ds(g * 16, 16)] = acc
        return 0

    lax.fori_loop(0, BPW // 16, group_body, 0)

    pltpu.sync_copy(outv, out_h.at[pl.ds(base, BPW)])


@jax.jit
def kernel(user, item_i, item_j, U, I):
    user3 = (user.astype(jnp.int32)).reshape(NW, NCHUNK, CHUNK)
    item_i3 = (item_i.astype(jnp.int32)).reshape(NW, NCHUNK, CHUNK)
    item_j3 = (item_j.astype(jnp.int32)).reshape(NW, NCHUNK, CHUNK)

    mesh = plsc.VectorSubcoreMesh(core_axis_name="c", subcore_axis_name="s")
    f = functools.partial(
        pl.kernel,
        out_type=jax.ShapeDtypeStruct((BATCH,), jnp.float32),
        mesh=mesh,
        compiler_params=pltpu.CompilerParams(
            needs_layout_passes=False, use_tc_tiling_on_sc=False
        ),
        scratch_types=[
            pltpu.VMEM((NCHUNK, CHUNK), jnp.int32),
            pltpu.VMEM((NCHUNK, CHUNK), jnp.int32),
            pltpu.VMEM((NCHUNK, CHUNK), jnp.int32),
            pltpu.VMEM((BPW, DIM), jnp.float32),
            pltpu.VMEM((BPW, DIM), jnp.float32),
            pltpu.VMEM((BPW, DIM), jnp.float32),
            pltpu.VMEM((BPW,), jnp.float32),
            pltpu.SemaphoreType.DMA,
        ],
    )(_bpr_body)
    return f(user3, item_i3, item_j3, U, I)
